# baseline (speedup 1.0000x reference)
import functools

import jax
import jax.numpy as jnp
from jax import lax
from jax.experimental import pallas as pl
from jax.experimental.pallas import tpu as pltpu
from jax.experimental.pallas import tpu_sc as plsc

NUM_CLASSES = 1000000
HIDDEN = 64
WORDS = HIDDEN // 2
BATCH = 16384
HIST = 200
TOTAL = BATCH * HIST
NW = 32
PER_W = TOTAL // NW
IDXW = 128
GPC = 8
CHUNK = IDXW * GPC
NITER = PER_W // CHUNK


def _sc_gather(table_i32, labels_rows):
    mesh = plsc.VectorSubcoreMesh(core_axis_name="c", subcore_axis_name="s")

    @functools.partial(
        pl.kernel,
        mesh=mesh,
        out_type=jax.ShapeDtypeStruct((TOTAL, WORDS), jnp.int32),
        scratch_types=[
            pltpu.VMEM((GPC, IDXW), jnp.int32),
            pltpu.VMEM((CHUNK, WORDS), jnp.int32),
            pltpu.SemaphoreType.DMA,
        ],
        compiler_params=pltpu.CompilerParams(use_tc_tiling_on_sc=False),
    )
    def k(table_hbm, labels_hbm, out_hbm, idx_v, rows_v, sem):
        nc = 2
        wid = lax.axis_index("s") * nc + lax.axis_index("c")
        base = wid * PER_W

        def body(g, carry):
            off = base + g * CHUNK
            row0 = pl.multiple_of(off // IDXW, 8)
            pltpu.sync_copy(labels_hbm.at[pl.ds(row0, GPC)], idx_v)
            handles = []
            for j in range(GPC):
                handles.append(pltpu.async_copy(
                    table_hbm.at[idx_v.at[j]],
                    rows_v.at[pl.ds(j * IDXW, IDXW)],
                    sem,
                ))
            for h in handles:
                h.wait()
            pltpu.sync_copy(rows_v, out_hbm.at[pl.ds(off, CHUNK)])
            return carry

        lax.fori_loop(0, NITER, body, 0)

    return k(table_i32, labels_rows)


def kernel(labels, embedding_table):
    table_bf16 = embedding_table.astype(jnp.bfloat16)
    table_i32 = lax.bitcast_convert_type(
        table_bf16.reshape(NUM_CLASSES, WORDS, 2), jnp.int32)
    labels_rows = labels.reshape(TOTAL // IDXW, IDXW).astype(jnp.int32)
    out_i32 = _sc_gather(table_i32, labels_rows)
    out_bf16 = lax.bitcast_convert_type(out_i32, jnp.bfloat16)
    return out_bf16.reshape(BATCH, HIST, HIDDEN)


# speedup vs baseline: 1.0115x; 1.0115x over previous
import functools

import jax
import jax.numpy as jnp
from jax import lax
from jax.experimental import pallas as pl
from jax.experimental.pallas import tpu as pltpu
from jax.experimental.pallas import tpu_sc as plsc

NUM_CLASSES = 1000000
HIDDEN = 64
WORDS = HIDDEN // 2
BATCH = 16384
HIST = 200
TOTAL = BATCH * HIST
NW = 32
PER_W = TOTAL // NW
IDXW = 128
D = 4
CH = 512
GP = CH // IDXW
NCH = PER_W // CH
IDX_ROWS = PER_W // IDXW


def _sc_gather(table_i32, labels_rows):
    mesh = plsc.VectorSubcoreMesh(core_axis_name="c", subcore_axis_name="s")

    scratch = ([pltpu.VMEM((GP, IDXW), jnp.int32) for _ in range(D)]
               + [pltpu.VMEM((CH, WORDS), jnp.int32) for _ in range(D)]
               + [pltpu.SemaphoreType.DMA for _ in range(2 * D)])

    @functools.partial(
        pl.kernel,
        mesh=mesh,
        out_type=jax.ShapeDtypeStruct((TOTAL, WORDS), jnp.int32),
        scratch_types=scratch,
        compiler_params=pltpu.CompilerParams(use_tc_tiling_on_sc=False),
    )
    def k(table_hbm, labels_hbm, out_hbm, *bufs):
        idx = bufs[0:D]
        rows = bufs[D:2 * D]
        gsem = bufs[2 * D:3 * D]
        ssem = bufs[3 * D:4 * D]

        nc = 2
        wid = lax.axis_index("s") * nc + lax.axis_index("c")
        base = wid * PER_W
        idx_base = wid * IDX_ROWS

        def load_idx(c, b):
            row0 = pl.multiple_of(idx_base + c * GP, GP)
            pltpu.sync_copy(labels_hbm.at[pl.ds(row0, GP)], idx[b])

        def fire_gathers(b):
            for j in range(GP):
                pltpu.async_copy(
                    table_hbm.at[idx[b].at[j]],
                    rows[b].at[pl.ds(j * IDXW, IDXW)],
                    gsem[b])

        def wait_gathers(b):
            for j in range(GP):
                pltpu.make_async_copy(
                    table_hbm.at[idx[b].at[j]],
                    rows[b].at[pl.ds(j * IDXW, IDXW)],
                    gsem[b]).wait()

        def fire_store(c, b):
            off = pl.multiple_of(base + c * CH, CH)
            pltpu.async_copy(rows[b], out_hbm.at[pl.ds(off, CH)], ssem[b])

        def wait_store(c, b):
            off = pl.multiple_of(base + c * CH, CH)
            pltpu.make_async_copy(rows[b], out_hbm.at[pl.ds(off, CH)],
                                  ssem[b]).wait()

        load_idx(0, 0)
        fire_gathers(0)
        load_idx(1, 1)
        fire_gathers(1)
        for c in (0, 1):
            wait_gathers(c)
            fire_store(c, c)
            load_idx(c + 2, c + 2)
            fire_gathers(c + 2)

        def body(g, carry):
            for u in range(4):
                c = 2 + g * 4 + u
                b = (2 + u) % 4
                b2 = u
                wait_gathers(b)
                fire_store(c, b)
                load_idx(c + 2, b2)
                wait_store(c - 2, b2)
                fire_gathers(b2)
            return carry

        lax.fori_loop(0, (NCH - 4) // 4, body, 0)

        for c in (NCH - 2, NCH - 1):
            b = c % 4
            wait_gathers(b)
            fire_store(c, b)
        for c in (NCH - 4, NCH - 3, NCH - 2, NCH - 1):
            wait_store(c, c % 4)

    return k(table_i32, labels_rows)


def kernel(labels, embedding_table):
    table_bf16 = embedding_table.astype(jnp.bfloat16)
    table_i32 = lax.bitcast_convert_type(
        table_bf16.reshape(NUM_CLASSES, WORDS, 2), jnp.int32)
    labels_rows = labels.reshape(TOTAL // IDXW, IDXW).astype(jnp.int32)
    out_i32 = _sc_gather(table_i32, labels_rows)
    out_bf16 = lax.bitcast_convert_type(out_i32, jnp.bfloat16)
    return out_bf16.reshape(BATCH, HIST, HIDDEN)


# speedup vs baseline: 2.5564x; 2.5273x over previous
import functools

import jax
import jax.numpy as jnp
from jax import lax
from jax.experimental import pallas as pl
from jax.experimental.pallas import tpu as pltpu
from jax.experimental.pallas import tpu_sc as plsc

NUM_CLASSES = 1000000
HIDDEN = 64
BATCH = 16384
HIST = 200
NW = 32
W_ROWS = BATCH // NW
R = 2
D = 4
NCH = W_ROWS // R
SPLITS = ((0, 104), (104, 96))


def _sc_gather(table_bf16, labels):
    mesh = plsc.VectorSubcoreMesh(core_axis_name="c", subcore_axis_name="s")

    scratch = ([pltpu.VMEM((R, HIST), jnp.int32) for _ in range(D)]
               + [pltpu.VMEM((R, HIST, HIDDEN), jnp.bfloat16) for _ in range(D)]
               + [pltpu.SemaphoreType.DMA for _ in range(2 * D)])

    @functools.partial(
        pl.kernel,
        mesh=mesh,
        out_type=jax.ShapeDtypeStruct((BATCH, HIST, HIDDEN), jnp.bfloat16),
        scratch_types=scratch,
        compiler_params=pltpu.CompilerParams(use_tc_tiling_on_sc=False),
    )
    def k(table_hbm, labels_hbm, out_hbm, *bufs):
        idx = bufs[0:D]
        rows = bufs[D:2 * D]
        gsem = bufs[2 * D:3 * D]
        ssem = bufs[3 * D:4 * D]

        nc = 2
        wid = lax.axis_index("s") * nc + lax.axis_index("c")
        base = wid * W_ROWS

        def load_idx(c, b):
            row0 = pl.multiple_of(base + c * R, R)
            pltpu.sync_copy(labels_hbm.at[pl.ds(row0, R)], idx[b])

        def fire_gathers(b):
            for r in range(R):
                for o, n in SPLITS:
                    pltpu.async_copy(
                        table_hbm.at[idx[b].at[r, pl.ds(o, n)]],
                        rows[b].at[r, pl.ds(o, n)],
                        gsem[b])

        def wait_gathers(b):
            for r in range(R):
                for o, n in SPLITS:
                    pltpu.make_async_copy(
                        table_hbm.at[idx[b].at[r, pl.ds(o, n)]],
                        rows[b].at[r, pl.ds(o, n)],
                        gsem[b]).wait()

        def fire_store(c, b):
            row0 = pl.multiple_of(base + c * R, R)
            pltpu.async_copy(rows[b], out_hbm.at[pl.ds(row0, R)], ssem[b])

        def wait_store(c, b):
            row0 = pl.multiple_of(base + c * R, R)
            pltpu.make_async_copy(rows[b], out_hbm.at[pl.ds(row0, R)],
                                  ssem[b]).wait()

        load_idx(0, 0)
        fire_gathers(0)
        load_idx(1, 1)
        fire_gathers(1)
        for c in (0, 1):
            wait_gathers(c)
            fire_store(c, c)
            load_idx(c + 2, c + 2)
            fire_gathers(c + 2)

        def body(g, carry):
            for u in range(4):
                c = 2 + g * 4 + u
                b = (2 + u) % 4
                b2 = u
                wait_gathers(b)
                fire_store(c, b)
                load_idx(c + 2, b2)
                wait_store(c - 2, b2)
                fire_gathers(b2)
            return carry

        lax.fori_loop(0, (NCH - 4) // 4, body, 0)

        for c in (NCH - 2, NCH - 1):
            b = c % 4
            wait_gathers(b)
            fire_store(c, b)
        for c in (NCH - 4, NCH - 3, NCH - 2, NCH - 1):
            wait_store(c, c % 4)

    return k(table_bf16, labels)


def kernel(labels, embedding_table):
    table_bf16 = embedding_table.astype(jnp.bfloat16)
    return _sc_gather(table_bf16, labels.astype(jnp.int32))


# speedup vs baseline: 2.9756x; 1.1640x over previous
import functools

import jax
import jax.numpy as jnp
from jax import lax
from jax.experimental import pallas as pl
from jax.experimental.pallas import tpu as pltpu
from jax.experimental.pallas import tpu_sc as plsc

NUM_CLASSES = 1000000
HIDDEN = 64
BATCH = 16384
HIST = 200
NW = 32
W_ROWS = BATCH // NW
R = 2
D = 4
NCH = W_ROWS // R
SPLITS = ((0, 104), (104, 96))


def _sc_gather(table_bf16, labels):
    mesh = plsc.VectorSubcoreMesh(core_axis_name="c", subcore_axis_name="s")

    scratch = ([pltpu.VMEM((R, HIST), jnp.int32) for _ in range(D)]
               + [pltpu.VMEM((R, HIST, HIDDEN // 2), jnp.int32) for _ in range(D)]
               + [pltpu.SemaphoreType.DMA for _ in range(2 * D)])

    @functools.partial(
        pl.kernel,
        mesh=mesh,
        out_type=jax.ShapeDtypeStruct((BATCH, HIST, HIDDEN // 2), jnp.int32),
        scratch_types=scratch,
        compiler_params=pltpu.CompilerParams(use_tc_tiling_on_sc=False),
    )
    def k(table_hbm, labels_hbm, out_hbm, *bufs):
        idx = bufs[0:D]
        rows = bufs[D:2 * D]
        gsem = bufs[2 * D:3 * D]
        ssem = bufs[3 * D:4 * D]

        nc = 2
        wid = lax.axis_index("s") * nc + lax.axis_index("c")
        base = wid * W_ROWS

        def load_idx(c, b):
            row0 = pl.multiple_of(base + c * R, R)
            pltpu.sync_copy(labels_hbm.at[pl.ds(row0, R)], idx[b])

        def fire_gathers(b):
            for r in range(R):
                for o, n in SPLITS:
                    pltpu.async_copy(
                        table_hbm.at[idx[b].at[r, pl.ds(o, n)]],
                        rows[b].at[r, pl.ds(o, n)],
                        gsem[b])

        def wait_gathers(b):
            for r in range(R):
                for o, n in SPLITS:
                    pltpu.make_async_copy(
                        table_hbm.at[idx[b].at[r, pl.ds(o, n)]],
                        rows[b].at[r, pl.ds(o, n)],
                        gsem[b]).wait()

        def fire_store(c, b):
            row0 = pl.multiple_of(base + c * R, R)
            pltpu.async_copy(rows[b], out_hbm.at[pl.ds(row0, R)], ssem[b])

        def wait_store(c, b):
            row0 = pl.multiple_of(base + c * R, R)
            pltpu.make_async_copy(rows[b], out_hbm.at[pl.ds(row0, R)],
                                  ssem[b]).wait()

        load_idx(0, 0)
        fire_gathers(0)
        load_idx(1, 1)
        fire_gathers(1)
        for c in (0, 1):
            wait_gathers(c)
            fire_store(c, c)
            load_idx(c + 2, c + 2)
            fire_gathers(c + 2)

        def body(g, carry):
            for u in range(4):
                c = 2 + g * 4 + u
                b = (2 + u) % 4
                b2 = u
                wait_gathers(b)
                fire_store(c, b)
                load_idx(c + 2, b2)
                wait_store(c - 2, b2)
                fire_gathers(b2)
            return carry

        lax.fori_loop(0, (NCH - 4) // 4, body, 0)

        for c in (NCH - 2, NCH - 1):
            b = c % 4
            wait_gathers(b)
            fire_store(c, b)
        for c in (NCH - 4, NCH - 3, NCH - 2, NCH - 1):
            wait_store(c, c % 4)

    return k(table_bf16, labels)


TB = 128
TG = BATCH // TB


QW = HIST * HIDDEN // 2 // 128


def _tc_format(out_i32):
    def body(x_ref, o_ref):
        for q in range(QW):
            xq = x_ref[pl.Slice(q, TB, QW), :]
            xqt = lax.transpose(xq, (1, 0))
            u = lax.bitcast_convert_type(xqt, jnp.uint32)
            lo = lax.bitcast_convert_type(
                (u & 0xffff).astype(jnp.uint16), jnp.bfloat16)
            hi = lax.bitcast_convert_type(
                (u >> 16).astype(jnp.uint16), jnp.bfloat16)
            for ch in range(4):
                l = lo[ch * 32:(ch + 1) * 32, :]
                h = hi[ch * 32:(ch + 1) * 32, :]
                blk = jnp.stack([l, h], axis=1)
                o_ref[4 * q + ch, :, :] = blk.reshape(HIDDEN, TB)

    flat = out_i32.reshape(BATCH * QW, 128)
    return pl.pallas_call(
        body,
        grid=(TG,),
        in_specs=[pl.BlockSpec((TB * QW, 128), lambda i: (i, 0))],
        out_specs=pl.BlockSpec((HIST, HIDDEN, TB), lambda i: (0, 0, i)),
        out_shape=jax.ShapeDtypeStruct((HIST, HIDDEN, BATCH), jnp.bfloat16),
    )(flat)


def kernel(labels, embedding_table):
    table_bf16 = embedding_table.astype(jnp.bfloat16)
    table_i32 = lax.bitcast_convert_type(
        table_bf16.reshape(NUM_CLASSES, HIDDEN // 2, 2), jnp.int32)
    out_i32 = _sc_gather(table_i32, labels.astype(jnp.int32))
    out_t = _tc_format(out_i32)
    return jnp.transpose(out_t, (2, 0, 1))


# speedup vs baseline: 3.9218x; 1.3180x over previous
import functools

import jax
import jax.numpy as jnp
from jax import lax
from jax.experimental import pallas as pl
from jax.experimental.pallas import tpu as pltpu
from jax.experimental.pallas import tpu_sc as plsc

NUM_CLASSES = 1000000
HIDDEN = 64
BATCH = 16384
HIST = 200
NW = 32
W_ROWS = BATCH // NW
R = 2
D = 4
NCH = W_ROWS // R
SPLITS = ((0, 104), (104, 96))


def _sc_gather(table_bf16, labels):
    mesh = plsc.VectorSubcoreMesh(core_axis_name="c", subcore_axis_name="s")

    scratch = ([pltpu.VMEM((R, HIST), jnp.int32) for _ in range(D)]
               + [pltpu.VMEM((R, HIST, HIDDEN // 2), jnp.int32) for _ in range(D)]
               + [pltpu.SemaphoreType.DMA for _ in range(2 * D)])

    @functools.partial(
        pl.kernel,
        mesh=mesh,
        out_type=jax.ShapeDtypeStruct((BATCH, HIST, HIDDEN // 2), jnp.int32),
        scratch_types=scratch,
        compiler_params=pltpu.CompilerParams(use_tc_tiling_on_sc=False),
    )
    def k(table_hbm, labels_hbm, out_hbm, *bufs):
        idx = bufs[0:D]
        rows = bufs[D:2 * D]
        gsem = bufs[2 * D:3 * D]
        ssem = bufs[3 * D:4 * D]

        nc = 2
        wid = lax.axis_index("s") * nc + lax.axis_index("c")
        base = wid * W_ROWS

        def load_idx(c, b):
            row0 = pl.multiple_of(base + c * R, R)
            pltpu.sync_copy(labels_hbm.at[pl.ds(row0, R)], idx[b])

        def fire_gathers(b):
            for r in range(R):
                for o, n in SPLITS:
                    pltpu.async_copy(
                        table_hbm.at[idx[b].at[r, pl.ds(o, n)]],
                        rows[b].at[r, pl.ds(o, n)],
                        gsem[b])

        def wait_gathers(b):
            for r in range(R):
                for o, n in SPLITS:
                    pltpu.make_async_copy(
                        table_hbm.at[idx[b].at[r, pl.ds(o, n)]],
                        rows[b].at[r, pl.ds(o, n)],
                        gsem[b]).wait()

        def fire_store(c, b):
            row0 = pl.multiple_of(base + c * R, R)
            pltpu.async_copy(rows[b], out_hbm.at[pl.ds(row0, R)], ssem[b])

        def wait_store(c, b):
            row0 = pl.multiple_of(base + c * R, R)
            pltpu.make_async_copy(rows[b], out_hbm.at[pl.ds(row0, R)],
                                  ssem[b]).wait()

        load_idx(0, 0)
        fire_gathers(0)
        load_idx(1, 1)
        fire_gathers(1)
        for c in (0, 1):
            wait_gathers(c)
            fire_store(c, c)
            load_idx(c + 2, c + 2)
            fire_gathers(c + 2)

        def body(g, carry):
            for u in range(4):
                c = 2 + g * 4 + u
                b = (2 + u) % 4
                b2 = u
                wait_gathers(b)
                fire_store(c, b)
                load_idx(c + 2, b2)
                wait_store(c - 2, b2)
                fire_gathers(b2)
            return carry

        lax.fori_loop(0, (NCH - 4) // 4, body, 0)

        for c in (NCH - 2, NCH - 1):
            b = c % 4
            wait_gathers(b)
            fire_store(c, b)
        for c in (NCH - 4, NCH - 3, NCH - 2, NCH - 1):
            wait_store(c, c % 4)

    return k(table_bf16, labels)


TB = 128
TG = BATCH // TB


QW = HIST * HIDDEN // 2 // 128


def _tc_format(out_i32):
    def body(x_ref, o_ref):
        for q in range(QW):
            xq = x_ref[pl.Slice(q, TB, QW), :]
            xqt = lax.transpose(xq, (1, 0))
            for ch in range(4):
                w = xqt[ch * 32:(ch + 1) * 32, :]
                o_ref[4 * q + ch, :, :] = pltpu.bitcast(w, jnp.bfloat16)

    flat = out_i32.reshape(BATCH * QW, 128)
    return pl.pallas_call(
        body,
        grid=(TG,),
        in_specs=[pl.BlockSpec((TB * QW, 128), lambda i: (i, 0))],
        out_specs=pl.BlockSpec((HIST, HIDDEN, TB), lambda i: (0, 0, i)),
        out_shape=jax.ShapeDtypeStruct((HIST, HIDDEN, BATCH), jnp.bfloat16),
    )(flat)


def kernel(labels, embedding_table):
    table_bf16 = embedding_table.astype(jnp.bfloat16)
    table_i32 = lax.bitcast_convert_type(
        table_bf16.reshape(NUM_CLASSES, HIDDEN // 2, 2), jnp.int32)
    out_i32 = _sc_gather(table_i32, labels.astype(jnp.int32))
    out_t = _tc_format(out_i32)
    return jnp.transpose(out_t, (2, 0, 1))


# speedup vs baseline: 10.0316x; 2.5579x over previous
import functools

import jax
import jax.numpy as jnp
from jax import lax
from jax.experimental import pallas as pl
from jax.experimental.pallas import tpu as pltpu
from jax.experimental.pallas import tpu_sc as plsc

NUM_CLASSES = 1000000
HIDDEN = 64
BATCH = 16384
HIST = 200
NW = 32
W_ROWS = BATCH // NW
R = 2
D = 4
NCH = W_ROWS // R
SPLITS = ((0, 104), (104, 96))


def _sc_gather(table_bf16, labels):
    mesh = plsc.VectorSubcoreMesh(core_axis_name="c", subcore_axis_name="s")

    scratch = ([pltpu.VMEM((R, HIST), jnp.int32) for _ in range(D)]
               + [pltpu.VMEM((R, HIST, HIDDEN // 2), jnp.int32) for _ in range(D)]
               + [pltpu.SemaphoreType.DMA for _ in range(2 * D)])

    @functools.partial(
        pl.kernel,
        mesh=mesh,
        out_type=jax.ShapeDtypeStruct((BATCH, HIST, HIDDEN // 2), jnp.int32),
        scratch_types=scratch,
        compiler_params=pltpu.CompilerParams(use_tc_tiling_on_sc=False),
    )
    def k(table_hbm, labels_hbm, out_hbm, *bufs):
        idx = bufs[0:D]
        rows = bufs[D:2 * D]
        gsem = bufs[2 * D:3 * D]
        ssem = bufs[3 * D:4 * D]

        nc = 2
        wid = lax.axis_index("s") * nc + lax.axis_index("c")
        base = wid * W_ROWS

        def load_idx(c, b):
            row0 = pl.multiple_of(base + c * R, R)
            pltpu.sync_copy(labels_hbm.at[pl.ds(row0, R)], idx[b])

        def fire_gathers(b):
            for r in range(R):
                for o, n in SPLITS:
                    pltpu.async_copy(
                        table_hbm.at[idx[b].at[r, pl.ds(o, n)]],
                        rows[b].at[r, pl.ds(o, n)],
                        gsem[b])

        def wait_gathers(b):
            for r in range(R):
                for o, n in SPLITS:
                    pltpu.make_async_copy(
                        table_hbm.at[idx[b].at[r, pl.ds(o, n)]],
                        rows[b].at[r, pl.ds(o, n)],
                        gsem[b]).wait()

        def fire_store(c, b):
            row0 = pl.multiple_of(base + c * R, R)
            pltpu.async_copy(rows[b], out_hbm.at[pl.ds(row0, R)], ssem[b])

        def wait_store(c, b):
            row0 = pl.multiple_of(base + c * R, R)
            pltpu.make_async_copy(rows[b], out_hbm.at[pl.ds(row0, R)],
                                  ssem[b]).wait()

        load_idx(0, 0)
        fire_gathers(0)
        load_idx(1, 1)
        fire_gathers(1)
        for c in (0, 1):
            wait_gathers(c)
            fire_store(c, c)
            load_idx(c + 2, c + 2)
            fire_gathers(c + 2)

        def body(g, carry):
            for u in range(4):
                c = 2 + g * 4 + u
                b = (2 + u) % 4
                b2 = u
                wait_gathers(b)
                fire_store(c, b)
                load_idx(c + 2, b2)
                wait_store(c - 2, b2)
                fire_gathers(b2)
            return carry

        lax.fori_loop(0, (NCH - 4) // 4, body, 0)

        for c in (NCH - 2, NCH - 1):
            b = c % 4
            wait_gathers(b)
            fire_store(c, b)
        for c in (NCH - 4, NCH - 3, NCH - 2, NCH - 1):
            wait_store(c, c % 4)

    return k(table_bf16, labels)


TB = 128
TG = BATCH // TB


QW = HIST * HIDDEN // 2 // 128


def _tc_format(out_i32):
    def body(x_ref, o_ref):
        for q in range(QW):
            xq = x_ref[pl.Slice(q, TB, QW), :]
            xqt = lax.transpose(xq, (1, 0))
            for ch in range(4):
                w = xqt[ch * 32:(ch + 1) * 32, :]
                o_ref[4 * q + ch, :, :] = pltpu.bitcast(w, jnp.bfloat16)

    flat = out_i32.reshape(BATCH * QW, 128)
    return pl.pallas_call(
        body,
        grid=(TG,),
        in_specs=[pl.BlockSpec((TB * QW, 128), lambda i: (i, 0))],
        out_specs=pl.BlockSpec((HIST, HIDDEN, TB), lambda i: (0, 0, i)),
        out_shape=jax.ShapeDtypeStruct((HIST, HIDDEN, BATCH), jnp.bfloat16),
    )(flat)


CB = 8192
TROWS = NUM_CLASSES * (HIDDEN // 2) // 128


def _tc_table(table_t):
    def body(x_ref, o_ref, scr_ref):
        x = x_ref[...]
        xb = x.astype(jnp.bfloat16)
        w = pltpu.bitcast(xb, jnp.int32)
        scr_ref[...] = lax.transpose(w, (1, 0))
        for rm in range(4):
            o_ref[:, rm * 32:(rm + 1) * 32] = scr_ref[pl.Slice(rm, CB // 4, 4), :]

    grid = (NUM_CLASSES + CB - 1) // CB
    return pl.pallas_call(
        body,
        grid=(grid,),
        in_specs=[pl.BlockSpec((HIDDEN, CB), lambda i: (0, i))],
        out_specs=pl.BlockSpec((CB // 4, 128), lambda i: (i, 0)),
        out_shape=jax.ShapeDtypeStruct((TROWS, 128), jnp.int32),
        scratch_shapes=[pltpu.VMEM((CB, 32), jnp.int32)],
    )(table_t)


def kernel(labels, embedding_table):
    table_i32 = _tc_table(embedding_table.T).reshape(NUM_CLASSES, HIDDEN // 2)
    out_i32 = _sc_gather(table_i32, labels.astype(jnp.int32))
    out_t = _tc_format(out_i32)
    return jnp.transpose(out_t, (2, 0, 1))


# speedup vs baseline: 10.3147x; 1.0282x over previous
import functools

import jax
import jax.numpy as jnp
from jax import lax
from jax.experimental import pallas as pl
from jax.experimental.pallas import tpu as pltpu
from jax.experimental.pallas import tpu_sc as plsc

NUM_CLASSES = 1000000
HIDDEN = 64
BATCH = 16384
HIST = 200
NW = 32
W_ROWS = BATCH // NW
R = 4
D = 4
NCH = W_ROWS // R
SPLITS = ((0, 104), (104, 96))


def _sc_gather(table_bf16, labels):
    mesh = plsc.VectorSubcoreMesh(core_axis_name="c", subcore_axis_name="s")

    scratch = ([pltpu.VMEM((R, HIST), jnp.int32) for _ in range(D)]
               + [pltpu.VMEM((R, HIST, HIDDEN // 2), jnp.int32) for _ in range(D)]
               + [pltpu.SemaphoreType.DMA for _ in range(2 * D)])

    @functools.partial(
        pl.kernel,
        mesh=mesh,
        out_type=jax.ShapeDtypeStruct((BATCH, HIST, HIDDEN // 2), jnp.int32),
        scratch_types=scratch,
        compiler_params=pltpu.CompilerParams(use_tc_tiling_on_sc=False),
    )
    def k(table_hbm, labels_hbm, out_hbm, *bufs):
        idx = bufs[0:D]
        rows = bufs[D:2 * D]
        gsem = bufs[2 * D:3 * D]
        ssem = bufs[3 * D:4 * D]

        nc = 2
        wid = lax.axis_index("s") * nc + lax.axis_index("c")
        base = wid * W_ROWS

        def load_idx(c, b):
            row0 = pl.multiple_of(base + c * R, R)
            pltpu.sync_copy(labels_hbm.at[pl.ds(row0, R)], idx[b])

        def fire_gathers(b):
            for r in range(R):
                for o, n in SPLITS:
                    pltpu.async_copy(
                        table_hbm.at[idx[b].at[r, pl.ds(o, n)]],
                        rows[b].at[r, pl.ds(o, n)],
                        gsem[b])

        def wait_gathers(b):
            for r in range(R):
                for o, n in SPLITS:
                    pltpu.make_async_copy(
                        table_hbm.at[idx[b].at[r, pl.ds(o, n)]],
                        rows[b].at[r, pl.ds(o, n)],
                        gsem[b]).wait()

        def fire_store(c, b):
            row0 = pl.multiple_of(base + c * R, R)
            pltpu.async_copy(rows[b], out_hbm.at[pl.ds(row0, R)], ssem[b])

        def wait_store(c, b):
            row0 = pl.multiple_of(base + c * R, R)
            pltpu.make_async_copy(rows[b], out_hbm.at[pl.ds(row0, R)],
                                  ssem[b]).wait()

        load_idx(0, 0)
        fire_gathers(0)
        load_idx(1, 1)
        fire_gathers(1)
        for c in (0, 1):
            wait_gathers(c)
            fire_store(c, c)
            load_idx(c + 2, c + 2)
            fire_gathers(c + 2)

        def body(g, carry):
            for u in range(4):
                c = 2 + g * 4 + u
                b = (2 + u) % 4
                b2 = u
                wait_gathers(b)
                fire_store(c, b)
                load_idx(c + 2, b2)
                wait_store(c - 2, b2)
                fire_gathers(b2)
            return carry

        lax.fori_loop(0, (NCH - 4) // 4, body, 0)

        for c in (NCH - 2, NCH - 1):
            b = c % 4
            wait_gathers(b)
            fire_store(c, b)
        for c in (NCH - 4, NCH - 3, NCH - 2, NCH - 1):
            wait_store(c, c % 4)

    return k(table_bf16, labels)


TB = 128
TG = BATCH // TB


QW = HIST * HIDDEN // 2 // 128


def _tc_format(out_i32):
    def body(x_ref, o_ref):
        for q in range(QW):
            xq = x_ref[pl.Slice(q, TB, QW), :]
            xqt = lax.transpose(xq, (1, 0))
            for ch in range(4):
                w = xqt[ch * 32:(ch + 1) * 32, :]
                o_ref[4 * q + ch, :, :] = pltpu.bitcast(w, jnp.bfloat16)

    flat = out_i32.reshape(BATCH * QW, 128)
    return pl.pallas_call(
        body,
        grid=(TG,),
        in_specs=[pl.BlockSpec((TB * QW, 128), lambda i: (i, 0))],
        out_specs=pl.BlockSpec((HIST, HIDDEN, TB), lambda i: (0, 0, i)),
        out_shape=jax.ShapeDtypeStruct((HIST, HIDDEN, BATCH), jnp.bfloat16),
    )(flat)


CB = 8192
TROWS = NUM_CLASSES * (HIDDEN // 2) // 128


def _tc_table(table_t):
    def body(x_ref, o_ref, scr_ref):
        x = x_ref[...]
        xb = x.astype(jnp.bfloat16)
        w = pltpu.bitcast(xb, jnp.int32)
        scr_ref[...] = lax.transpose(w, (1, 0))
        for rm in range(4):
            o_ref[:, rm * 32:(rm + 1) * 32] = scr_ref[pl.Slice(rm, CB // 4, 4), :]

    grid = (NUM_CLASSES + CB - 1) // CB
    return pl.pallas_call(
        body,
        grid=(grid,),
        in_specs=[pl.BlockSpec((HIDDEN, CB), lambda i: (0, i))],
        out_specs=pl.BlockSpec((CB // 4, 128), lambda i: (i, 0)),
        out_shape=jax.ShapeDtypeStruct((TROWS, 128), jnp.int32),
        scratch_shapes=[pltpu.VMEM((CB, 32), jnp.int32)],
    )(table_t)


def kernel(labels, embedding_table):
    table_i32 = _tc_table(embedding_table.T).reshape(NUM_CLASSES, HIDDEN // 2)
    out_i32 = _sc_gather(table_i32, labels.astype(jnp.int32))
    out_t = _tc_format(out_i32)
    return jnp.transpose(out_t, (2, 0, 1))
